# baseline (speedup 1.0000x reference)
import functools

import jax
import jax.numpy as jnp
import numpy as np
from jax.experimental import pallas as pl
from jax.experimental.pallas import tpu as pltpu

_NPTS = 64
_LANES = 128
_RQ = 56


def _make_kernel_body(h: int, w: int, rq: int):
    inv_w = np.float32(1.0 / w)
    col_step = np.float32(2.0 / (w - 1))
    row_step = np.float32(2.0 / (h - 1))
    k_step = np.float32(1.0 / (_NPTS - 1))

    def body(params_ref, depth_ref, xyp_ref, w_ref, cm_ref,
             zs_ref, dirs_ref, org_ref, xy_ref):
        fx = params_ref[0, 0, 0]
        fy = params_ref[0, 0, 1]
        px = params_ref[0, 0, 2]
        py = params_ref[0, 0, 3]
        r = [[params_ref[0, 0, 4 + 3 * i + j] for j in range(3)]
             for i in range(3)]
        t = [params_ref[0, 0, 13 + j] for j in range(3)]

        q0 = pl.program_id(0) * rq

        d_blk = depth_ref[0]
        ia = jax.lax.broadcasted_iota(jnp.int32, (_NPTS, _LANES), 0)
        il = jax.lax.broadcasted_iota(jnp.int32, (_NPTS, _LANES), 1)
        even_sel = (il == 2 * ia).astype(jnp.float32)
        odd_sel = (il == 2 * ia + 1).astype(jnp.float32)
        cdims = (((1,), (1,)), ((), ()))
        de_t = jax.lax.dot_general(even_sel, d_blk, cdims,
                                   preferred_element_type=jnp.float32)
        do_t = jax.lax.dot_general(odd_sel, d_blk, cdims,
                                   preferred_element_type=jnp.float32)
        lane1 = jax.lax.broadcasted_iota(jnp.int32, (1, _LANES), 1)
        lo_half = lane1 < _NPTS
        kline2 = ((lane1 % _NPTS).astype(jnp.float32) * k_step
                  + jnp.float32(0.5))
        for i in range(rq):
            dsel = jnp.where(lo_half, de_t[:, i:i + 1], do_t[:, i:i + 1])
            zs_ref[0, i] = dsel * kline2

        rowi = jax.lax.broadcasted_iota(jnp.int32, (rq, _LANES), 0)
        lane = jax.lax.broadcasted_iota(jnp.int32, (rq, _LANES), 1)
        pf = ((q0 + rowi) * _LANES + lane).astype(jnp.float32)
        qf = jnp.floor((pf + jnp.float32(0.5)) * inv_w)
        cf = pf - jnp.float32(w) * qf
        x = cf * col_step - jnp.float32(1.0)
        y = qf * row_step - jnp.float32(1.0)

        ux = (x - px) / fx
        uy = (y - py) / fy
        dx = ux * r[0][0] + uy * r[0][1] + r[0][2]
        dy = ux * r[1][0] + uy * r[1][1] + r[1][2]
        dz = ux * r[2][0] + uy * r[2][1] + r[2][2]
        inv = jax.lax.rsqrt(dx * dx + dy * dy + dz * dz)
        s_cat = jnp.concatenate([dx * inv, dy * inv, dz * inv],
                                axis=1)
        for s in range(3):
            dirs_ref[0, :, s, :] = jnp.dot(s_cat, w_ref[s],
                                           preferred_element_type=jnp.float32)

        o = [-(t[0] * r[i][0] + t[1] * r[i][1] + t[2] * r[i][2])
             for i in range(3)]
        for s in range(3):
            cms = cm_ref[s:s + 1, :]
            row = jnp.where(cms == 0.0, o[0],
                            jnp.where(cms == 1.0, o[1], o[2]))
            org_ref[0, :, s, :] = jnp.broadcast_to(row, (rq, _LANES))

        xy_ref[0] = xyp_ref[...]

    return body


@functools.cache
def _spread_select_w():
    w = np.zeros((3, 3 * _LANES, _LANES), np.float32)
    for s in range(3):
        for l in range(_LANES):
            m = _LANES * s + l
            c, p = m % 3, m // 3
            w[s, c * _LANES + p, l] = 1.0
    return jnp.asarray(w)


@functools.cache
def _cyc3_pattern():
    cm = np.zeros((3, _LANES), np.float32)
    for s in range(3):
        for l in range(_LANES):
            cm[s, l] = (_LANES * s + l) % 3
    return jnp.asarray(cm)


@jax.jit
def _run(depth_channel, R, T, focal, principal, xy_grid):
    B_, H_, W_ = depth_channel.shape
    n = H_ * W_
    nq = n // _LANES
    rq = _RQ
    grid = (nq // rq, B_)

    depth_in = depth_channel.reshape(B_, nq, _LANES)
    xy_pairs = xy_grid.reshape(2 * nq, _LANES)
    params = jnp.concatenate(
        [focal, principal, R.reshape(B_, 9), T], axis=1).reshape(B_, 1, 16)
    w_mat = _spread_select_w()
    cm = _cyc3_pattern()

    zs, dirs, org, xy = pl.pallas_call(
        _make_kernel_body(H_, W_, rq),
        grid=grid,
        in_specs=[
            pl.BlockSpec((1, 1, 16), lambda q, b: (b, 0, 0),
                         memory_space=pltpu.SMEM),
            pl.BlockSpec((1, rq, _LANES), lambda q, b: (b, q, 0)),
            pl.BlockSpec((2 * rq, _LANES), lambda q, b: (q, 0)),
            pl.BlockSpec((3, 3 * _LANES, _LANES), lambda q, b: (0, 0, 0)),
            pl.BlockSpec((3, _LANES), lambda q, b: (0, 0)),
        ],
        out_specs=[
            pl.BlockSpec((1, rq, _NPTS, _LANES), lambda q, b: (b, q, 0, 0)),
            pl.BlockSpec((1, rq, 3, _LANES), lambda q, b: (b, q, 0, 0)),
            pl.BlockSpec((1, rq, 3, _LANES), lambda q, b: (b, q, 0, 0)),
            pl.BlockSpec((1, 2 * rq, _LANES), lambda q, b: (b, q, 0)),
        ],
        out_shape=[
            jax.ShapeDtypeStruct((B_, nq, _NPTS, _LANES), jnp.float32),
            jax.ShapeDtypeStruct((B_, nq, 3, _LANES), jnp.float32),
            jax.ShapeDtypeStruct((B_, nq, 3, _LANES), jnp.float32),
            jax.ShapeDtypeStruct((B_, 2 * nq, _LANES), jnp.float32),
        ],
        compiler_params=pltpu.CompilerParams(
            dimension_semantics=("arbitrary", "arbitrary")),
    )(params, depth_in, xy_pairs, w_mat, cm)

    return (org.reshape(B_, H_, W_, 3),
            dirs.reshape(B_, H_, W_, 3),
            zs.reshape(B_, H_, W_, _NPTS),
            xy.reshape(B_, H_, W_, 2))


def kernel(depth_channel, R, T, focal, principal, xy_grid):
    return _run(depth_channel, R, T, focal, principal, xy_grid)


# speedup vs baseline: 19.3288x; 19.3288x over previous
import jax
import jax.numpy as jnp
import numpy as np
from jax.experimental import pallas as pl
from jax.experimental.pallas import tpu as pltpu

_NPTS = 64
_RH = 112


def _make_kernel_body(h: int, w: int, rh: int):
    col_step = np.float32(2.0 / (w - 1))
    row_step = np.float32(2.0 / (h - 1))
    k_step = np.float32(1.0 / (_NPTS - 1))

    def body(params_ref, depth_ref, zs_ref, dirs_ref, org_ref):
        fx = params_ref[0, 0, 0]
        fy = params_ref[0, 0, 1]
        px = params_ref[0, 0, 2]
        py = params_ref[0, 0, 3]
        r = [[params_ref[0, 0, 4 + 3 * i + j] for j in range(3)]
             for i in range(3)]
        t = [params_ref[0, 0, 13 + j] for j in range(3)]

        d_blk = depth_ref[0]
        line_col = (jax.lax.broadcasted_iota(jnp.int32, (_NPTS, 1), 0)
                    .astype(jnp.float32) * k_step + jnp.float32(0.5))
        line_pl = jnp.broadcast_to(line_col, (_NPTS, w))
        for i in range(rh):
            zs_ref[0, i] = line_pl * d_blk[i:i + 1, :]

        h0 = pl.program_id(0) * rh
        x_row = (jax.lax.broadcasted_iota(jnp.int32, (1, w), 1)
                 .astype(jnp.float32) * col_step - jnp.float32(1.0))
        y_col = ((h0 + jax.lax.broadcasted_iota(jnp.int32, (rh, 1), 0)
                  .astype(jnp.float32)) * row_step - jnp.float32(1.0))
        ux = (x_row - px) / fx
        uy = (y_col - py) / fy
        dx = ux * r[0][0] + uy * r[0][1] + r[0][2]
        dy = ux * r[1][0] + uy * r[1][1] + r[1][2]
        dz = ux * r[2][0] + uy * r[2][1] + r[2][2]
        ss = dx * dx + dy * dy + dz * dz
        inv = jax.lax.rsqrt(ss)
        inv = inv * (jnp.float32(1.5) - jnp.float32(0.5) * ss * inv * inv)
        dirs_ref[0, 0] = dx * inv
        dirs_ref[0, 1] = dy * inv
        dirs_ref[0, 2] = dz * inv

        for c in range(3):
            o_c = -(t[0] * r[c][0] + t[1] * r[c][1] + t[2] * r[c][2])
            org_ref[0, c] = jnp.full((rh, w), o_c, jnp.float32)

    return body


@jax.jit
def _run(depth_channel, R, T, focal, principal, xy_grid):
    B_, H_, W_ = depth_channel.shape
    rh = _RH
    grid = (H_ // rh, B_)

    params = jnp.concatenate(
        [focal, principal, R.reshape(B_, 9), T], axis=1).reshape(B_, 1, 16)

    zs_p, dirs_p, org_p = pl.pallas_call(
        _make_kernel_body(H_, W_, rh),
        grid=grid,
        in_specs=[
            pl.BlockSpec((1, 1, 16), lambda q, b: (b, 0, 0),
                         memory_space=pltpu.SMEM),
            pl.BlockSpec((1, rh, W_), lambda q, b: (b, q, 0)),
        ],
        out_specs=[
            pl.BlockSpec((1, rh, _NPTS, W_), lambda q, b: (b, q, 0, 0)),
            pl.BlockSpec((1, 3, rh, W_), lambda q, b: (b, 0, q, 0)),
            pl.BlockSpec((1, 3, rh, W_), lambda q, b: (b, 0, q, 0)),
        ],
        out_shape=[
            jax.ShapeDtypeStruct((B_, H_, _NPTS, W_), jnp.float32),
            jax.ShapeDtypeStruct((B_, 3, H_, W_), jnp.float32),
            jax.ShapeDtypeStruct((B_, 3, H_, W_), jnp.float32),
        ],
        compiler_params=pltpu.CompilerParams(
            dimension_semantics=("parallel", "parallel")),
    )(params, depth_channel)

    return (jnp.transpose(org_p, (0, 2, 3, 1)),
            jnp.transpose(dirs_p, (0, 2, 3, 1)),
            jnp.transpose(zs_p, (0, 1, 3, 2)),
            jnp.broadcast_to(xy_grid[None], (B_, H_, W_, 2)))


def kernel(depth_channel, R, T, focal, principal, xy_grid):
    return _run(depth_channel, R, T, focal, principal, xy_grid)
